# baseline (speedup 1.0000x reference)
import functools

import jax
import jax.numpy as jnp
import numpy as np
from jax import lax
from jax.experimental import pallas as pl
from jax.experimental.pallas import tpu as pltpu
from jax.experimental.pallas import tpu_sc as plsc

VOCAB = 100000
NFIELD = 26
EMB = 16
B = 4096
TOT = B * NFIELD
SROW = NFIELD * VOCAB // 8
OROW = TOT // 8

NC = 2
NS = 16
NW = NC * NS
PER_W = TOT // NW
CHUNK = 128
NCHUNK = PER_W // CHUNK
NBUF = 2
OROW_W = PER_W // 8

_BN = float(1.0 / np.sqrt(1.0 + 1e-5))


def _sc_gather(srow_idx, sub_idx, flat_idx, emb2_sr, emb1_flat):
    mesh = plsc.VectorSubcoreMesh(core_axis_name="c", subcore_axis_name="s")

    @functools.partial(
        pl.kernel,
        mesh=mesh,
        compiler_params=pltpu.CompilerParams(
            use_tc_tiling_on_sc=True, needs_layout_passes=False),
        out_type=(
            jax.ShapeDtypeStruct((OROW, 128), jnp.float32),
            jax.ShapeDtypeStruct((TOT,), jnp.float32),
        ),
        scratch_types=[
            pltpu.VMEM((PER_W,), jnp.int32),
            pltpu.VMEM((PER_W,), jnp.int32),
            pltpu.VMEM((PER_W,), jnp.int32),
            pltpu.VMEM((CHUNK, 128), jnp.float32),
            pltpu.VMEM((CHUNK, 128), jnp.float32),
            pltpu.VMEM((OROW_W, 128), jnp.float32),
            pltpu.VMEM((PER_W,), jnp.float32),
            pltpu.SemaphoreType.DMA,
            pltpu.SemaphoreType.DMA,
        ],
    )
    def k(srow_hbm, sub_hbm, flat_hbm, e2_hbm, e1_hbm, out2, out1,
          idx_v, sub_v, flat_v, gbuf0, gbuf1, obuf, g1_v, sem2, sem1):
        wid = lax.axis_index("s") * NC + lax.axis_index("c")
        base = wid * PER_W
        pltpu.sync_copy(srow_hbm.at[pl.ds(base, PER_W)], idx_v)
        pltpu.sync_copy(sub_hbm.at[pl.ds(base, PER_W)], sub_v)
        pltpu.sync_copy(flat_hbm.at[pl.ds(base, PER_W)], flat_v)
        gbufs = (gbuf0, gbuf1)

        def csl(c):
            return pl.ds(pl.multiple_of(c * CHUNK, 8), CHUNK)

        def fire2(c, b):
            pltpu.async_copy(e2_hbm.at[idx_v.at[csl(c)]], gbufs[b], sem2)

        def wait2(b):
            pltpu.make_async_copy(e2_hbm.at[idx_v.at[csl(0)]], gbufs[b], sem2).wait()

        def fire1(c):
            pltpu.async_copy(e1_hbm.at[flat_v.at[csl(c)]], g1_v.at[csl(c)], sem1)

        def wait1():
            pltpu.make_async_copy(
                e1_hbm.at[flat_v.at[csl(0)]], g1_v.at[csl(0)], sem1).wait()

        lane = jnp.arange(16, dtype=jnp.int32)

        def extract(c, b):
            gb = gbufs[b]
            for i16 in range(8):
                j0 = c * CHUNK + i16 * 16
                sub16 = sub_v[pl.ds(pl.multiple_of(j0, 8), 16)]
                grow = lane + i16 * 16
                orow = (j0 >> 3) + (lane >> 3)
                for e in range(16):
                    v = plsc.load_gather(gb, [grow, sub16 + e])
                    plsc.store_scatter(obuf, [orow, (lane & 7) * 16 + e], v)

        fire2(0, 0)
        fire2(1, 1)
        fire1(0)
        fire1(1)

        def body(i, carry):
            for b in range(NBUF):
                c = i * NBUF + b
                wait2(b)
                wait1()
                extract(c, b)

                @pl.when(c + NBUF < NCHUNK)
                def _():
                    fire2(c + NBUF, b)
                    fire1(c + NBUF)

            return carry

        lax.fori_loop(0, NCHUNK // NBUF, body, 0)
        pltpu.sync_copy(obuf, out2.at[pl.ds(wid * OROW_W, OROW_W)])
        pltpu.sync_copy(g1_v, out1.at[pl.ds(base, PER_W)])

    return k(srow_idx, sub_idx, flat_idx, emb2_sr, emb1_flat)


BB = 512
NBLK = B // BB


def _tc_body(
    go2_ref, go1_ref, xv_ref, s_ref, t_ref,
    w1_ref, b1_ref, w2_ref, b2_ref, w3_ref, b3_ref,
    ga1_ref, be1_ref, ga2_ref, be2_ref, ga3_ref, be3_ref, bias_ref, out_ref,
):
    xv_w = xv_ref[...]
    xv = go2_ref[...] * jnp.dot(xv_w, s_ref[...], preferred_element_type=jnp.float32)
    s = jnp.dot(xv, t_ref[...], preferred_element_type=jnp.float32)
    f2 = 0.5 * (jnp.sum(s * s, axis=1) - jnp.sum(xv * xv, axis=1))
    f1 = jnp.sum(go1_ref[...] * xv_w, axis=1)
    h = jnp.maximum(jnp.dot(xv, w1_ref[...], preferred_element_type=jnp.float32) + b1_ref[...], 0.0)
    h = h * (ga1_ref[...] * _BN) + be1_ref[...]
    h = jnp.maximum(jnp.dot(h, w2_ref[...], preferred_element_type=jnp.float32) + b2_ref[...], 0.0)
    h = h * (ga2_ref[...] * _BN) + be2_ref[...]
    h = jnp.maximum(jnp.dot(h, w3_ref[...], preferred_element_type=jnp.float32) + b3_ref[...], 0.0)
    h = h * (ga3_ref[...] * _BN) + be3_ref[...]
    out_ref[...] = (f1 + f2 + jnp.sum(h, axis=1) + bias_ref[0]).reshape(1, 1, BB)


def _tc_call(go2, go1, Xv, S, T, w1t, b1, w2t, b2, w3t, b3,
             ga1, be1, ga2, be2, ga3, be3, bias):
    full = lambda shape: pl.BlockSpec(shape, lambda i: (0, 0))
    out = pl.pallas_call(
        _tc_body,
        grid=(NBLK,),
        in_specs=[
            pl.BlockSpec((BB, NFIELD * EMB), lambda i: (i, 0)),
            pl.BlockSpec((BB, NFIELD), lambda i: (i, 0)),
            pl.BlockSpec((BB, NFIELD), lambda i: (i, 0)),
            full((NFIELD, NFIELD * EMB)),
            full((NFIELD * EMB, EMB)),
            full((NFIELD * EMB, 32)),
            full((1, 32)),
            full((32, 32)),
            full((1, 32)),
            full((32, 32)),
            full((1, 32)),
            full((1, 32)),
            full((1, 32)),
            full((1, 32)),
            full((1, 32)),
            full((1, 32)),
            full((1, 32)),
            pl.BlockSpec(memory_space=pltpu.SMEM),
        ],
        out_specs=pl.BlockSpec((1, 1, BB), lambda i: (i, 0, 0)),
        out_shape=jax.ShapeDtypeStruct((NBLK, 1, BB), jnp.float32),
    )(go2, go1, Xv, S, T, w1t, b1, w2t, b2, w3t, b3,
      ga1, be1, ga2, be2, ga3, be3, bias)
    return out.reshape(B)


def kernel(Xi, Xv, emb1, emb2, W1, b1, W2, b2, W3, b3, g1, be1, g2, be2, g3, be3, bias):
    idx = Xi[:, :, 0].astype(jnp.int32)
    flat = idx + (jnp.arange(NFIELD, dtype=jnp.int32) * VOCAB)[None, :]
    flat = flat.reshape(TOT)
    srow = flat // 8
    sub = (flat % 8) * EMB
    e2sr = emb2.reshape(SROW, 128)
    e1f = emb1.reshape(NFIELD * VOCAB)
    go2, go1 = _sc_gather(srow, sub, flat, e2sr, e1f)
    go2 = go2.reshape(B, NFIELD * EMB)
    go1 = go1.reshape(B, NFIELD)

    eye_f = jnp.eye(NFIELD, dtype=jnp.float32)
    S = jnp.repeat(eye_f, EMB, axis=1)
    T = jnp.tile(jnp.eye(EMB, dtype=jnp.float32), (NFIELD, 1))

    r = lambda v: v.reshape(1, -1)
    return _tc_call(
        go2, go1, Xv, S, T,
        W1.T, r(b1), W2.T, r(b2), W3.T, r(b3),
        r(g1), r(be1), r(g2), r(be2), r(g3), r(be3), bias,
    )


# speedup vs baseline: 1.0271x; 1.0271x over previous
import functools

import jax
import jax.numpy as jnp
import numpy as np
from jax import lax
from jax.experimental import pallas as pl
from jax.experimental.pallas import tpu as pltpu
from jax.experimental.pallas import tpu_sc as plsc

VOCAB = 100000
NFIELD = 26
EMB = 16
B = 4096
TOT = B * NFIELD

NC = 2
NS = 16
NW = NC * NS
PER_W = TOT // NW
CHUNK = 128
NCHUNK = PER_W // CHUNK
NPIPE = 4

_BN = float(1.0 / np.sqrt(1.0 + 1e-5))


def _sc_gather(flat_idx, emb2_flat, emb1_flat):
    mesh = plsc.VectorSubcoreMesh(core_axis_name="c", subcore_axis_name="s")

    @functools.partial(
        pl.kernel,
        mesh=mesh,
        compiler_params=pltpu.CompilerParams(use_tc_tiling_on_sc=False),
        out_type=(
            jax.ShapeDtypeStruct((TOT, EMB), jnp.float32),
            jax.ShapeDtypeStruct((TOT,), jnp.float32),
        ),
        scratch_types=[
            pltpu.VMEM((PER_W,), jnp.int32),
            pltpu.VMEM((PER_W, EMB), jnp.float32),
            pltpu.VMEM((PER_W,), jnp.float32),
            pltpu.SemaphoreType.DMA,
            pltpu.SemaphoreType.DMA,
        ],
    )
    def k(idx_hbm, e2_hbm, e1_hbm, out2, out1, idx_v, g2_v, g1_v, sem2, sem1):
        wid = lax.axis_index("s") * NC + lax.axis_index("c")
        base = wid * PER_W
        pltpu.sync_copy(idx_hbm.at[pl.ds(base, PER_W)], idx_v)

        def chunk_refs(c):
            sl = pl.ds(pl.multiple_of(c * CHUNK, 8), CHUNK)
            return (e2_hbm.at[idx_v.at[sl]], g2_v.at[sl]), (
                e1_hbm.at[idx_v.at[sl]], g1_v.at[sl])

        def fire(c):
            (s2, d2), (s1, d1) = chunk_refs(c)
            pltpu.async_copy(s2, d2, sem2)
            pltpu.async_copy(s1, d1, sem1)

        def wait_one():
            (s2, d2), (s1, d1) = chunk_refs(0)
            pltpu.make_async_copy(s2, d2, sem2).wait()
            pltpu.make_async_copy(s1, d1, sem1).wait()

        for c in range(NPIPE):
            fire(c)

        def body(c, carry):
            fire(c)
            wait_one()
            return carry

        lax.fori_loop(NPIPE, NCHUNK, body, 0)
        for _ in range(NPIPE):
            wait_one()
        pltpu.sync_copy(g2_v, out2.at[pl.ds(base, PER_W)])
        pltpu.sync_copy(g1_v, out1.at[pl.ds(base, PER_W)])

    return k(flat_idx, emb2_flat, emb1_flat)


BB = 512
NBLK = B // BB


def _tc_body(
    go2_ref, go1_ref, xv_ref, s_ref, t_ref,
    w1_ref, b1_ref, w2_ref, b2_ref, w3_ref, b3_ref,
    ga1_ref, be1_ref, ga2_ref, be2_ref, ga3_ref, be3_ref, bias_ref, out_ref,
):
    xv_w = xv_ref[...]
    xv = go2_ref[...] * jnp.dot(xv_w, s_ref[...], preferred_element_type=jnp.float32)
    s = jnp.dot(xv, t_ref[...], preferred_element_type=jnp.float32)
    f2 = 0.5 * (jnp.sum(s * s, axis=1) - jnp.sum(xv * xv, axis=1))
    f1 = jnp.sum(go1_ref[...] * xv_w, axis=1)
    h = jnp.maximum(jnp.dot(xv, w1_ref[...], preferred_element_type=jnp.float32) + b1_ref[...], 0.0)
    h = h * (ga1_ref[...] * _BN) + be1_ref[...]
    h = jnp.maximum(jnp.dot(h, w2_ref[...], preferred_element_type=jnp.float32) + b2_ref[...], 0.0)
    h = h * (ga2_ref[...] * _BN) + be2_ref[...]
    h = jnp.maximum(jnp.dot(h, w3_ref[...], preferred_element_type=jnp.float32) + b3_ref[...], 0.0)
    h = h * (ga3_ref[...] * _BN) + be3_ref[...]
    out_ref[...] = (f1 + f2 + jnp.sum(h, axis=1) + bias_ref[0]).reshape(1, 1, BB)


def _tc_call(go2, go1, Xv, S, T, w1t, b1, w2t, b2, w3t, b3,
             ga1, be1, ga2, be2, ga3, be3, bias):
    full = lambda shape: pl.BlockSpec(shape, lambda i: (0, 0))
    out = pl.pallas_call(
        _tc_body,
        grid=(NBLK,),
        in_specs=[
            pl.BlockSpec((BB, NFIELD * EMB), lambda i: (i, 0)),
            pl.BlockSpec((BB, NFIELD), lambda i: (i, 0)),
            pl.BlockSpec((BB, NFIELD), lambda i: (i, 0)),
            full((NFIELD, NFIELD * EMB)),
            full((NFIELD * EMB, EMB)),
            full((NFIELD * EMB, 32)),
            full((1, 32)),
            full((32, 32)),
            full((1, 32)),
            full((32, 32)),
            full((1, 32)),
            full((1, 32)),
            full((1, 32)),
            full((1, 32)),
            full((1, 32)),
            full((1, 32)),
            full((1, 32)),
            pl.BlockSpec(memory_space=pltpu.SMEM),
        ],
        out_specs=pl.BlockSpec((1, 1, BB), lambda i: (i, 0, 0)),
        out_shape=jax.ShapeDtypeStruct((NBLK, 1, BB), jnp.float32),
    )(go2, go1, Xv, S, T, w1t, b1, w2t, b2, w3t, b3,
      ga1, be1, ga2, be2, ga3, be3, bias)
    return out.reshape(B)


def kernel(Xi, Xv, emb1, emb2, W1, b1, W2, b2, W3, b3, g1, be1, g2, be2, g3, be3, bias):
    idx = Xi[:, :, 0].astype(jnp.int32)
    flat = idx + (jnp.arange(NFIELD, dtype=jnp.int32) * VOCAB)[None, :]
    flat = flat.reshape(TOT)
    e2f = emb2.reshape(NFIELD * VOCAB, EMB)
    e1f = emb1.reshape(NFIELD * VOCAB)
    go2, go1 = _sc_gather(flat, e2f, e1f)
    go2 = go2.reshape(B, NFIELD * EMB)
    go1 = go1.reshape(B, NFIELD)

    eye_f = jnp.eye(NFIELD, dtype=jnp.float32)
    S = jnp.repeat(eye_f, EMB, axis=1)
    T = jnp.tile(jnp.eye(EMB, dtype=jnp.float32), (NFIELD, 1))

    r = lambda v: v.reshape(1, -1)
    return _tc_call(
        go2, go1, Xv, S, T,
        W1.T, r(b1), W2.T, r(b2), W3.T, r(b3),
        r(g1), r(be1), r(g2), r(be2), r(g3), r(be3), bias,
    )
